# baseline (speedup 1.0000x reference)
import functools

import jax
import jax.numpy as jnp
from jax import lax
from jax.experimental import pallas as pl
from jax.experimental.pallas import tpu as pltpu
from jax.experimental.pallas import tpu_sc as plsc

B = 16384
D = 128
NC = 2
NS = 16
NW = NC * NS
BPW = B // NW
CH = 128
NCHUNK = BPW // CH
L = 16
GPC = CH // L
DU = 8


def _scores_kernel(users_hbm, pos_hbm, neg_hbm, ut_hbm, it_hbm,
                   pos_out, neg_out,
                   uidx, pidx, nidx,
                   ubuf0, pbuf0, nbuf0, ubuf1, pbuf1, nbuf1,
                   psc, nsc, sem0, sem1):
    wid = lax.axis_index("s") * NC + lax.axis_index("c")
    base = wid * BPW
    ubufs = (ubuf0, ubuf1)
    pbufs = (pbuf0, pbuf1)
    nbufs = (nbuf0, nbuf1)
    sems = (sem0, sem1)

    for c in range(NCHUNK):
        cbase = base + c * CH
        pltpu.sync_copy(users_hbm.at[pl.ds(cbase, CH)], uidx.at[c])
        pltpu.sync_copy(pos_hbm.at[pl.ds(cbase, CH)], pidx.at[c])
        pltpu.sync_copy(neg_hbm.at[pl.ds(cbase, CH)], nidx.at[c])

    def fire(c, s):
        return (pltpu.async_copy(ut_hbm.at[uidx.at[c]], ubufs[s], sems[s]),
                pltpu.async_copy(it_hbm.at[pidx.at[c]], pbufs[s], sems[s]),
                pltpu.async_copy(it_hbm.at[nidx.at[c]], nbufs[s], sems[s]))

    inflight = {0: fire(0, 0)}
    for c in range(NCHUNK):
        s = c % 2
        if c + 1 < NCHUNK:
            inflight[c + 1] = fire(c + 1, (c + 1) % 2)
        for h in inflight.pop(c):
            h.wait()
        ubuf, pbuf, nbuf = ubufs[s], pbufs[s], nbufs[s]

        lane_iota = lax.iota(jnp.int32, L)

        def gbody(g, carry, c=c):
            rows = lane_iota + g * L

            def dbody(i, acc, rows=rows):
                accp, accn = acc
                for j in range(DU):
                    col = jnp.zeros((L,), jnp.int32) + (i * DU + j)
                    uv = plsc.load_gather(ubuf, [rows, col])
                    pv = plsc.load_gather(pbuf, [rows, col])
                    nv = plsc.load_gather(nbuf, [rows, col])
                    accp = accp + uv * pv
                    accn = accn + uv * nv
                return accp, accn

            accp, accn = lax.fori_loop(
                0, D // DU, dbody,
                (jnp.zeros((L,), jnp.float32),
                 jnp.zeros((L,), jnp.float32)))
            psc[pl.ds(c * CH + g * L, L)] = accp
            nsc[pl.ds(c * CH + g * L, L)] = accn
            return carry

        lax.fori_loop(0, GPC, gbody, 0)

    pltpu.sync_copy(psc, pos_out.at[pl.ds(base, BPW)])
    pltpu.sync_copy(nsc, neg_out.at[pl.ds(base, BPW)])


@jax.jit
def _scores(users, posItems, negItems, embedUserTable, embedItemTable):
    mesh = plsc.VectorSubcoreMesh(core_axis_name="c", subcore_axis_name="s")
    run = functools.partial(
        pl.kernel,
        mesh=mesh,
        compiler_params=pltpu.CompilerParams(needs_layout_passes=False),
        out_type=(
            jax.ShapeDtypeStruct((B,), jnp.float32),
            jax.ShapeDtypeStruct((B,), jnp.float32),
        ),
        scratch_types=[
            pltpu.VMEM((NCHUNK, CH), jnp.int32),
            pltpu.VMEM((NCHUNK, CH), jnp.int32),
            pltpu.VMEM((NCHUNK, CH), jnp.int32),
            pltpu.VMEM((CH, D), jnp.float32),
            pltpu.VMEM((CH, D), jnp.float32),
            pltpu.VMEM((CH, D), jnp.float32),
            pltpu.VMEM((CH, D), jnp.float32),
            pltpu.VMEM((CH, D), jnp.float32),
            pltpu.VMEM((CH, D), jnp.float32),
            pltpu.VMEM((BPW,), jnp.float32),
            pltpu.VMEM((BPW,), jnp.float32),
            pltpu.SemaphoreType.DMA,
            pltpu.SemaphoreType.DMA,
        ],
    )(_scores_kernel)
    return run(users, posItems, negItems, embedUserTable, embedItemTable)


def kernel(users, seqs, posItems, negItems, embedUserTable, embedItemTable):
    del seqs
    return _scores(users.astype(jnp.int32), posItems.astype(jnp.int32),
                   negItems.astype(jnp.int32), embedUserTable, embedItemTable)


# speedup vs baseline: 2.8414x; 2.8414x over previous
import functools

import jax
import jax.numpy as jnp
from jax import lax
from jax.experimental import pallas as pl
from jax.experimental.pallas import tpu as pltpu
from jax.experimental.pallas import tpu_sc as plsc

B = 16384
D = 128
NC = 2
NS = 16
NW = NC * NS
BPW = B // NW
CH = 128
NCHUNK = BPW // CH
L = 16
RU = 8


def _scores_kernel(users_hbm, pos_hbm, neg_hbm, ut_hbm, it_hbm,
                   pos_out, neg_out,
                   uidx, pidx, nidx,
                   ubuf0, pbuf0, nbuf0, ubuf1, pbuf1, nbuf1,
                   psc, nsc, sem0, sem1):
    wid = lax.axis_index("s") * NC + lax.axis_index("c")
    base = wid * BPW
    ubufs = (ubuf0, ubuf1)
    pbufs = (pbuf0, pbuf1)
    nbufs = (nbuf0, nbuf1)
    sems = (sem0, sem1)

    for c in range(NCHUNK):
        cbase = base + c * CH
        pltpu.sync_copy(users_hbm.at[pl.ds(cbase, CH)], uidx.at[c])
        pltpu.sync_copy(pos_hbm.at[pl.ds(cbase, CH)], pidx.at[c])
        pltpu.sync_copy(neg_hbm.at[pl.ds(cbase, CH)], nidx.at[c])

    def fire(c, s):
        return (pltpu.async_copy(ut_hbm.at[uidx.at[c]], ubufs[s], sems[s]),
                pltpu.async_copy(it_hbm.at[pidx.at[c]], pbufs[s], sems[s]),
                pltpu.async_copy(it_hbm.at[nidx.at[c]], nbufs[s], sems[s]))

    inflight = {0: fire(0, 0)}
    for c in range(NCHUNK):
        s = c % 2
        if c + 1 < NCHUNK:
            inflight[c + 1] = fire(c + 1, (c + 1) % 2)
        for h in inflight.pop(c):
            h.wait()
        ubuf, pbuf, nbuf = ubufs[s], pbufs[s], nbufs[s]

        last_lane = lax.iota(jnp.int32, L) == (L - 1)

        def rblock(b, carry, c=c):
            for k in range(RU):
                r = b * RU + k
                u0 = ubuf[r, pl.ds(0, L)]
                ap = u0 * pbuf[r, pl.ds(0, L)]
                an = u0 * nbuf[r, pl.ds(0, L)]
                for j in range(1, D // L):
                    u = ubuf[r, pl.ds(j * L, L)]
                    ap = ap + u * pbuf[r, pl.ds(j * L, L)]
                    an = an + u * nbuf[r, pl.ds(j * L, L)]
                off = c * CH + r
                plsc.store_compressed(psc.at[pl.ds(off, L)],
                                      plsc.cumsum(ap), mask=last_lane)
                plsc.store_compressed(nsc.at[pl.ds(off, L)],
                                      plsc.cumsum(an), mask=last_lane)
            return carry

        lax.fori_loop(0, CH // RU, rblock, 0)

    pltpu.sync_copy(psc.at[pl.ds(0, BPW)], pos_out.at[pl.ds(base, BPW)])
    pltpu.sync_copy(nsc.at[pl.ds(0, BPW)], neg_out.at[pl.ds(base, BPW)])


@jax.jit
def _scores(users, posItems, negItems, embedUserTable, embedItemTable):
    mesh = plsc.VectorSubcoreMesh(core_axis_name="c", subcore_axis_name="s")
    run = functools.partial(
        pl.kernel,
        mesh=mesh,
        compiler_params=pltpu.CompilerParams(needs_layout_passes=False),
        out_type=(
            jax.ShapeDtypeStruct((B,), jnp.float32),
            jax.ShapeDtypeStruct((B,), jnp.float32),
        ),
        scratch_types=[
            pltpu.VMEM((NCHUNK, CH), jnp.int32),
            pltpu.VMEM((NCHUNK, CH), jnp.int32),
            pltpu.VMEM((NCHUNK, CH), jnp.int32),
            pltpu.VMEM((CH, D), jnp.float32),
            pltpu.VMEM((CH, D), jnp.float32),
            pltpu.VMEM((CH, D), jnp.float32),
            pltpu.VMEM((CH, D), jnp.float32),
            pltpu.VMEM((CH, D), jnp.float32),
            pltpu.VMEM((CH, D), jnp.float32),
            pltpu.VMEM((BPW + L,), jnp.float32),
            pltpu.VMEM((BPW + L,), jnp.float32),
            pltpu.SemaphoreType.DMA,
            pltpu.SemaphoreType.DMA,
        ],
    )(_scores_kernel)
    return run(users, posItems, negItems, embedUserTable, embedItemTable)


def kernel(users, seqs, posItems, negItems, embedUserTable, embedItemTable):
    del seqs
    return _scores(users.astype(jnp.int32), posItems.astype(jnp.int32),
                   negItems.astype(jnp.int32), embedUserTable, embedItemTable)


# speedup vs baseline: 3.2230x; 1.1343x over previous
import functools

import jax
import jax.numpy as jnp
from jax import lax
from jax.experimental import pallas as pl
from jax.experimental.pallas import tpu as pltpu
from jax.experimental.pallas import tpu_sc as plsc

B = 16384
D = 128
NC = 2
NS = 16
NW = NC * NS
BPW = B // NW
CH = 128
NCHUNK = BPW // CH
L = 16
RU = 4


def _scores_kernel(users_hbm, pos_hbm, neg_hbm, ut_hbm, it_hbm,
                   pos_out, neg_out,
                   uidx, pidx, nidx,
                   ubuf0, pbuf0, nbuf0, ubuf1, pbuf1, nbuf1,
                   psc, nsc, sem0, sem1):
    wid = lax.axis_index("s") * NC + lax.axis_index("c")
    base = wid * BPW
    ubufs = (ubuf0, ubuf1)
    pbufs = (pbuf0, pbuf1)
    nbufs = (nbuf0, nbuf1)
    sems = (sem0, sem1)

    h0 = (pltpu.async_copy(users_hbm.at[pl.ds(base, CH)],
                           uidx.at[pl.ds(0, CH)], sem0),
          pltpu.async_copy(pos_hbm.at[pl.ds(base, CH)],
                           pidx.at[pl.ds(0, CH)], sem0),
          pltpu.async_copy(neg_hbm.at[pl.ds(base, CH)],
                           nidx.at[pl.ds(0, CH)], sem0))
    rest = BPW - CH
    h1 = (pltpu.async_copy(users_hbm.at[pl.ds(base + CH, rest)],
                           uidx.at[pl.ds(CH, rest)], sem1),
          pltpu.async_copy(pos_hbm.at[pl.ds(base + CH, rest)],
                           pidx.at[pl.ds(CH, rest)], sem1),
          pltpu.async_copy(neg_hbm.at[pl.ds(base + CH, rest)],
                           nidx.at[pl.ds(CH, rest)], sem1))

    def fire(c, s):
        sl = pl.ds(c * CH, CH)
        return (pltpu.async_copy(ut_hbm.at[uidx.at[sl]], ubufs[s], sems[s]),
                pltpu.async_copy(it_hbm.at[pidx.at[sl]], pbufs[s], sems[s]),
                pltpu.async_copy(it_hbm.at[nidx.at[sl]], nbufs[s], sems[s]))

    for h in h0:
        h.wait()
    inflight = {0: fire(0, 0)}
    for h in h1:
        h.wait()
    for c in range(NCHUNK):
        s = c % 2
        if c + 1 < NCHUNK:
            inflight[c + 1] = fire(c + 1, (c + 1) % 2)
        for h in inflight.pop(c):
            h.wait()
        ubuf, pbuf, nbuf = ubufs[s], pbufs[s], nbufs[s]

        last_lane = lax.iota(jnp.int32, L) == (L - 1)

        def ablock(b):
            accs = []
            for k in range(RU):
                r = b * RU + k
                u0 = ubuf[r, pl.ds(0, L)]
                ap = u0 * pbuf[r, pl.ds(0, L)]
                an = u0 * nbuf[r, pl.ds(0, L)]
                for j in range(1, D // L):
                    u = ubuf[r, pl.ds(j * L, L)]
                    ap = ap + u * pbuf[r, pl.ds(j * L, L)]
                    an = an + u * nbuf[r, pl.ds(j * L, L)]
                accs += [ap, an]
            return tuple(accs)

        def finish(b, accs, c=c):
            for k in range(RU):
                off = c * CH + b * RU + k
                plsc.store_compressed(psc.at[pl.ds(off, L)],
                                      plsc.cumsum(accs[2 * k]),
                                      mask=last_lane)
                plsc.store_compressed(nsc.at[pl.ds(off, L)],
                                      plsc.cumsum(accs[2 * k + 1]),
                                      mask=last_lane)

        def rblock(b, carry):
            finish(b - 1, carry)
            return ablock(b)

        nblk = CH // RU
        tail = lax.fori_loop(1, nblk, rblock, ablock(0))
        finish(nblk - 1, tail)

    pltpu.sync_copy(psc.at[pl.ds(0, BPW)], pos_out.at[pl.ds(base, BPW)])
    pltpu.sync_copy(nsc.at[pl.ds(0, BPW)], neg_out.at[pl.ds(base, BPW)])


@jax.jit
def _scores(users, posItems, negItems, embedUserTable, embedItemTable):
    mesh = plsc.VectorSubcoreMesh(core_axis_name="c", subcore_axis_name="s")
    run = functools.partial(
        pl.kernel,
        mesh=mesh,
        compiler_params=pltpu.CompilerParams(needs_layout_passes=False),
        out_type=(
            jax.ShapeDtypeStruct((B,), jnp.float32),
            jax.ShapeDtypeStruct((B,), jnp.float32),
        ),
        scratch_types=[
            pltpu.VMEM((BPW,), jnp.int32),
            pltpu.VMEM((BPW,), jnp.int32),
            pltpu.VMEM((BPW,), jnp.int32),
            pltpu.VMEM((CH, D), jnp.float32),
            pltpu.VMEM((CH, D), jnp.float32),
            pltpu.VMEM((CH, D), jnp.float32),
            pltpu.VMEM((CH, D), jnp.float32),
            pltpu.VMEM((CH, D), jnp.float32),
            pltpu.VMEM((CH, D), jnp.float32),
            pltpu.VMEM((BPW + L,), jnp.float32),
            pltpu.VMEM((BPW + L,), jnp.float32),
            pltpu.SemaphoreType.DMA,
            pltpu.SemaphoreType.DMA,
        ],
    )(_scores_kernel)
    return run(users, posItems, negItems, embedUserTable, embedItemTable)


def kernel(users, seqs, posItems, negItems, embedUserTable, embedItemTable):
    del seqs
    return _scores(users.astype(jnp.int32), posItems.astype(jnp.int32),
                   negItems.astype(jnp.int32), embedUserTable, embedItemTable)
